# baseline (speedup 1.0000x reference)
import jax
import jax.numpy as jnp
from jax import lax
from jax.experimental import pallas as pl
from jax.experimental.pallas import tpu as pltpu

DEPTH_ = 1000
B_BLK = 16
NBUF = 8


def _onehot_block(idx_ref, out_hbm, bufs, sems):
    i = pl.program_id(0)
    nsteps = pl.num_programs(0)
    k = lax.rem(i, NBUF)

    @pl.when(i >= NBUF)
    def _():
        pltpu.make_async_copy(
            bufs.at[k], out_hbm.at[pl.ds((i - NBUF) * B_BLK, B_BLK)], sems.at[k]
        ).wait()

    vals = idx_ref[...]
    t = vals.shape[1]
    cols = lax.broadcasted_iota(jnp.int32, (B_BLK, t, DEPTH_), 2)
    v3 = vals[:, :, None]
    hit = (cols == v3) & (v3 != 0)
    bufs[k] = hit.astype(jnp.float32)

    pltpu.make_async_copy(
        bufs.at[k], out_hbm.at[pl.ds(i * B_BLK, B_BLK)], sems.at[k]
    ).start()

    @pl.when(i == nsteps - 1)
    def _():
        for j in range(NBUF):
            step = nsteps - NBUF + j
            kk = lax.rem(jnp.int32(step), NBUF)
            pltpu.make_async_copy(
                bufs.at[kk], out_hbm.at[pl.ds(step * B_BLK, B_BLK)], sems.at[kk]
            ).wait()


def kernel(inputs):
    b, t = inputs.shape
    out = pl.pallas_call(
        _onehot_block,
        grid=(b // B_BLK,),
        in_specs=[pl.BlockSpec((B_BLK, t), lambda i: (i, 0))],
        out_specs=pl.BlockSpec(memory_space=pl.ANY),
        out_shape=jax.ShapeDtypeStruct((b, t, DEPTH_), jnp.float32),
        scratch_shapes=[
            pltpu.VMEM((NBUF, B_BLK, t, DEPTH_), jnp.float32),
            pltpu.SemaphoreType.DMA((NBUF,)),
        ],
    )(inputs)
    return out
